# Initial kernel scaffold; baseline (speedup 1.0000x reference)
import jax
import jax.numpy as jnp
from jax.experimental import pallas as pl


def kernel(x, y, alphas, beta):
    raise NotImplementedError("write your pallas kernel here")


# speedup vs baseline: 2.4247x; 2.4247x over previous
import functools

import jax
import jax.numpy as jnp
from jax import lax
from jax.experimental import pallas as pl
from jax.experimental.pallas import tpu as pltpu
from jax.experimental.pallas import tpu_sc as plsc

N = 16384
NC, NS, L = 2, 16, 16
NW = NC * NS
CHUNK = N // NW
STEPS = CHUNK // L


@functools.partial(
    pl.kernel,
    mesh=plsc.VectorSubcoreMesh(core_axis_name="c", subcore_axis_name="s"),
    out_type=jax.ShapeDtypeStruct((N,), jnp.float32),
    compiler_params=pltpu.CompilerParams(needs_layout_passes=False),
    scratch_types=[
        pltpu.VMEM((2 * CHUNK,), jnp.float32),
        pltpu.VMEM((2 * CHUNK,), jnp.float32),
        pltpu.VMEM((64,), jnp.float32),
        pltpu.VMEM((8,), jnp.float32),
        pltpu.VMEM((CHUNK,), jnp.float32),
    ],
)
def _sc_kernel(x_hbm, y_hbm, alphas_hbm, beta_hbm, out_hbm,
               x_v, y_v, a_v, b_v, o_v):
    wid = lax.axis_index("s") * NC + lax.axis_index("c")
    base = wid * CHUNK
    pltpu.sync_copy(x_hbm.at[pl.ds(2 * base, 2 * CHUNK)], x_v)
    pltpu.sync_copy(y_hbm.at[pl.ds(2 * base, 2 * CHUNK)], y_v)
    pltpu.sync_copy(alphas_hbm, a_v)
    pltpu.sync_copy(beta_hbm, b_v)

    lane = lax.iota(jnp.int32, L)
    for j in range(STEPS):
        it = lane * 2 + (2 * L * j)
        ic = it + 1
        t_x = plsc.load_gather(x_v, [it])
        c_x = plsc.load_gather(x_v, [ic])
        t_y = plsc.load_gather(y_v, [it])
        c_y = plsc.load_gather(y_v, [ic])
        xi = c_x.astype(jnp.int32)
        yi = c_y.astype(jnp.int32)
        a = plsc.load_gather(a_v, [xi * 8 + yi])
        b = plsc.load_gather(b_v, [yi])
        mask = t_x > 0.0
        tds = jnp.where(mask, t_x - t_y, 0.0)
        o_v[pl.ds(j * L, L)] = jnp.where(mask, a * b * jnp.exp(-b * tds), 0.0)

    pltpu.sync_copy(o_v, out_hbm.at[pl.ds(base, CHUNK)])


def kernel(x, y, alphas, beta):
    return _sc_kernel(x.reshape(-1), y.reshape(-1), alphas.reshape(-1), beta)


# speedup vs baseline: 2.4997x; 1.0309x over previous
import functools

import jax
import jax.numpy as jnp
from jax import lax
from jax.experimental import pallas as pl
from jax.experimental.pallas import tpu as pltpu
from jax.experimental.pallas import tpu_sc as plsc

N = 16384
NC, NS, L = 2, 16, 16
NW = NC * NS
CHUNK = N // NW
STEPS = CHUNK // L


@functools.partial(
    pl.kernel,
    mesh=plsc.VectorSubcoreMesh(core_axis_name="c", subcore_axis_name="s"),
    out_type=jax.ShapeDtypeStruct((N,), jnp.float32),
    compiler_params=pltpu.CompilerParams(needs_layout_passes=False),
    scratch_types=[
        pltpu.VMEM((2 * CHUNK,), jnp.float32),
        pltpu.VMEM((2 * CHUNK,), jnp.float32),
        pltpu.VMEM((64,), jnp.float32),
        pltpu.VMEM((8,), jnp.float32),
        pltpu.VMEM((CHUNK,), jnp.float32),
        pltpu.SemaphoreType.DMA,
    ],
)
def _sc_kernel(x_hbm, y_hbm, alphas_hbm, beta_hbm, out_hbm,
               x_v, y_v, a_v, b_v, o_v, sem):
    wid = lax.axis_index("s") * NC + lax.axis_index("c")
    base = wid * CHUNK
    copies = [
        pltpu.make_async_copy(x_hbm.at[pl.ds(2 * base, 2 * CHUNK)], x_v, sem),
        pltpu.make_async_copy(y_hbm.at[pl.ds(2 * base, 2 * CHUNK)], y_v, sem),
        pltpu.make_async_copy(alphas_hbm, a_v, sem),
        pltpu.make_async_copy(beta_hbm, b_v, sem),
    ]
    for c in copies:
        c.start()
    for c in copies:
        c.wait()

    lane = lax.iota(jnp.int32, L)
    for j in range(STEPS):
        it = lane * 2 + (2 * L * j)
        ic = it + 1
        t_x = plsc.load_gather(x_v, [it])
        c_x = plsc.load_gather(x_v, [ic])
        t_y = plsc.load_gather(y_v, [it])
        c_y = plsc.load_gather(y_v, [ic])
        xi = c_x.astype(jnp.int32)
        yi = c_y.astype(jnp.int32)
        a = plsc.load_gather(a_v, [xi * 8 + yi])
        b = plsc.load_gather(b_v, [yi])
        mask = t_x > 0.0
        tds = jnp.where(mask, t_x - t_y, 0.0)
        o_v[pl.ds(j * L, L)] = jnp.where(mask, a * b * jnp.exp(-b * tds), 0.0)

    pltpu.sync_copy(o_v, out_hbm.at[pl.ds(base, CHUNK)])


def kernel(x, y, alphas, beta):
    return _sc_kernel(x.reshape(-1), y.reshape(-1), alphas.reshape(-1), beta)
